# baseline (speedup 1.0000x reference)
import functools

import jax
import jax.numpy as jnp
from jax import lax
from jax.experimental import pallas as pl
from jax.experimental.pallas import tpu as pltpu
from jax.experimental.pallas import tpu_sc as plsc

_CUTOFF = 0.09
_NW = 32
_LANES = 16
_BLK = 2048


def _sc_edge_gather(posx, posy, posz, typ, ei, n_elem):
    e_pad = ei.shape[1]
    chunk = e_pad // _NW
    cols = e_pad // 8
    steps = chunk // _LANES
    n_atoms = posx.shape[0]

    def body(posx_h, posy_h, posz_h, typ_h, ei_h, r2_h, pr_h,
             px, py, pz, ty, sv, dv, r2v, prv):
        cid = lax.axis_index("c")
        sid = lax.axis_index("s")
        wid = cid * 16 + sid
        base = wid * chunk
        row = wid // 4
        c0 = (wid % 4) * chunk
        pltpu.sync_copy(posx_h, px)
        pltpu.sync_copy(posy_h, py)
        pltpu.sync_copy(posz_h, pz)
        pltpu.sync_copy(typ_h, ty)
        pltpu.sync_copy(ei_h.at[0, pl.ds(base, chunk)], sv)
        pltpu.sync_copy(ei_h.at[1, pl.ds(base, chunk)], dv)

        @plsc.parallel_loop(0, steps, step=1, unroll=8)
        def _step(i):
            sl = pl.ds(i * _LANES, _LANES)
            s = sv[sl]
            d = dv[sl]
            xs = plsc.load_gather(px, [s])
            xd = plsc.load_gather(px, [d])
            ys = plsc.load_gather(py, [s])
            yd = plsc.load_gather(py, [d])
            zs = plsc.load_gather(pz, [s])
            zd = plsc.load_gather(pz, [d])
            ts = plsc.load_gather(ty, [s])
            td = plsc.load_gather(ty, [d])
            ddx = xd - xs
            ddy = yd - ys
            ddz = zd - zs
            r2v[sl] = ddx * ddx + ddy * ddy + ddz * ddz
            prv[sl] = td * n_elem + ts
        pltpu.sync_copy(r2v, r2_h.at[row, pl.ds(c0, chunk)])
        pltpu.sync_copy(prv, pr_h.at[row, pl.ds(c0, chunk)])

    mesh = plsc.VectorSubcoreMesh(core_axis_name="c", subcore_axis_name="s")
    fn = pl.kernel(
        body,
        mesh=mesh,
        compiler_params=pltpu.CompilerParams(needs_layout_passes=False),
        out_type=(
            jax.ShapeDtypeStruct((8, cols), jnp.float32),
            jax.ShapeDtypeStruct((8, cols), jnp.int32),
        ),
        scratch_types=[
            pltpu.VMEM((n_atoms,), jnp.float32),
            pltpu.VMEM((n_atoms,), jnp.float32),
            pltpu.VMEM((n_atoms,), jnp.float32),
            pltpu.VMEM((n_atoms,), jnp.int32),
            pltpu.VMEM((chunk,), jnp.int32),
            pltpu.VMEM((chunk,), jnp.int32),
            pltpu.VMEM((chunk,), jnp.float32),
            pltpu.VMEM((chunk,), jnp.int32),
        ],
    )
    return fn(posx, posy, posz, typ, ei)


_CB = 2048
_BMAX = 10


def _tc_energy(r2p, prp, w1k, w2k, b2k, w3k, b3c, n_edges, ne):
    cols = r2p.shape[1]
    grid = cols // _CB
    nfeat = _BMAX + 2 * ne

    def body(r2_ref, pr_ref, w1_ref, w2_ref, b2_ref, w3_ref, b3_ref, out_ref):
        pid = pl.program_id(0)

        @pl.when(pid == 0)
        def _():
            out_ref[0, 0] = 0.0

        r2 = r2_ref[...]
        pr = pr_ref[...]
        r = jnp.sqrt(r2)
        x = r * (1.0 / _CUTOFF)
        x2 = x * x
        x3 = x2 * x
        x6 = x3 * x3
        x7 = x6 * x
        x8 = x7 * x
        env = 1.0 - 28.0 * x6 + 48.0 * x7 - 21.0 * x8
        env = jnp.where(x < 1.0, env, 0.0)
        scale = jnp.sqrt(2.0 / _CUTOFF) * env / jnp.maximum(r, 1e-12)
        ang = jnp.pi * x
        c1 = jnp.cos(ang)
        two_c1 = c1 + c1
        s = [jnp.sin(ang) * scale]
        sm1 = jnp.zeros_like(r2)
        for _n in range(_BMAX - 1):
            s_next = two_c1 * s[-1] - sm1
            sm1 = s[-1]
            s.append(s_next)
        td = pr // ne
        ts = pr - td * ne
        oh = [(td == t).astype(jnp.float32) for t in range(ne)]
        oh += [(ts == t).astype(jnp.float32) for t in range(ne)]
        feat = jnp.concatenate(s + oh, axis=0)
        pre1 = jnp.dot(w1_ref[...], feat,
                       preferred_element_type=jnp.float32)
        h1 = pre1 * jax.nn.sigmoid(pre1)
        pre2 = jnp.dot(w2_ref[...], h1,
                       preferred_element_type=jnp.float32) + b2_ref[...]
        h2 = pre2 * jax.nn.sigmoid(pre2)
        evec = jnp.dot(w3_ref[...], h2,
                       preferred_element_type=jnp.float32)
        jrow = lax.broadcasted_iota(jnp.int32, (8, _CB), 0) * cols
        lane = lax.broadcasted_iota(jnp.int32, (8, _CB), 1)
        eid = jrow + pid * _CB + lane
        e = jnp.where(eid < n_edges, evec + b3_ref[0, 0], 0.0)
        out_ref[0, 0] += 0.5 * jnp.sum(e)

    out = pl.pallas_call(
        body,
        grid=(grid,),
        in_specs=[
            pl.BlockSpec((8, _CB), lambda i: (0, i)),
            pl.BlockSpec((8, _CB), lambda i: (0, i)),
            pl.BlockSpec((128, 8 * nfeat), lambda i: (0, 0)),
            pl.BlockSpec((128, 128), lambda i: (0, 0)),
            pl.BlockSpec((128, 1), lambda i: (0, 0)),
            pl.BlockSpec((8, 128), lambda i: (0, 0)),
            pl.BlockSpec(memory_space=pltpu.SMEM),
        ],
        out_specs=pl.BlockSpec(memory_space=pltpu.SMEM),
        out_shape=jax.ShapeDtypeStruct((1, 1), jnp.float32),
    )(r2p, prp, w1k, w2k, b2k, w3k, b3c)
    return out


def kernel(positions, type_indices, edge_index, emb_table, W1, b1, W2, b2, W3, b3):
    n_edges = edge_index.shape[1]
    blk_edges = 8 * _CB
    e_pad = ((n_edges + blk_edges - 1) // blk_edges) * blk_edges
    e_half = ((e_pad // 2 + blk_edges - 1) // blk_edges) * blk_edges
    if e_half >= n_edges:
        e_half = 0
    ei1 = edge_index[:, :e_half]
    ei2 = jnp.pad(edge_index[:, e_half:], ((0, 0), (0, e_pad - n_edges)))
    posx = positions[:, 0]
    posy = positions[:, 1]
    posz = positions[:, 2]

    ne = emb_table.shape[0]
    td_dim = emb_table.shape[1]
    hid = W1.shape[0]
    a_d = emb_table @ W1[:, :td_dim].T + b1
    b_s = emb_table @ W1[:, td_dim:2 * td_dim].T
    w18 = jnp.concatenate([W1[:, 2 * td_dim:], a_d.T, b_s.T], axis=1)
    eye8 = jnp.eye(8, dtype=jnp.float32)
    w1k = jnp.kron(w18, eye8)
    w2k = jnp.kron(W2, eye8)
    b2k = jnp.repeat(b2, 8).reshape(8 * hid, 1)
    w3k = jnp.kron(W3, eye8)

    b3c = b3.reshape(1, 1)
    if e_half:
        r2a, pra = _sc_edge_gather(posx, posy, posz, type_indices, ei1, ne)
        r2b, prb = _sc_edge_gather(posx, posy, posz, type_indices, ei2, ne)
        o1 = _tc_energy(r2a, pra, w1k, w2k, b2k, w3k, b3c, e_half, ne)
        o2 = _tc_energy(r2b, prb, w1k, w2k, b2k, w3k, b3c,
                        n_edges - e_half, ne)
        return o1[0, 0] + o2[0, 0]
    r2p, prp = _sc_edge_gather(posx, posy, posz, type_indices, ei2, ne)
    out = _tc_energy(r2p, prp, w1k, w2k, b2k, w3k, b3c, n_edges, ne)
    return out[0, 0]


# speedup vs baseline: 1.0730x; 1.0730x over previous
import functools

import jax
import jax.numpy as jnp
from jax import lax
from jax.experimental import pallas as pl
from jax.experimental.pallas import tpu as pltpu
from jax.experimental.pallas import tpu_sc as plsc

_CUTOFF = 0.09
_NW = 32
_LANES = 16
_BLK = 2048


def _sc_edge_gather(posx, posy, posz, typ, ei, n_elem):
    e_pad = ei.shape[1]
    chunk = e_pad // _NW
    cols = e_pad // 8
    steps = chunk // _LANES
    n_atoms = posx.shape[0]

    def body(posx_h, posy_h, posz_h, typ_h, ei_h, r2_h, pr_h,
             px, py, pz, ty, sv, dv, r2v, prv, sem):
        cid = lax.axis_index("c")
        sid = lax.axis_index("s")
        wid = cid * 16 + sid
        base = wid * chunk
        row = wid // 4
        c0 = (wid % 4) * chunk
        h1 = pltpu.async_copy(posx_h, px, sem)
        h2 = pltpu.async_copy(posy_h, py, sem)
        h3 = pltpu.async_copy(posz_h, pz, sem)
        h4 = pltpu.async_copy(typ_h, ty, sem)
        h5 = pltpu.async_copy(ei_h.at[0, pl.ds(base, chunk)], sv, sem)
        h6 = pltpu.async_copy(ei_h.at[1, pl.ds(base, chunk)], dv, sem)
        h1.wait()
        h2.wait()
        h3.wait()
        h4.wait()
        h5.wait()
        h6.wait()

        @plsc.parallel_loop(0, steps, step=1, unroll=8)
        def _step(i):
            sl = pl.ds(i * _LANES, _LANES)
            s = sv[sl]
            d = dv[sl]
            xs = plsc.load_gather(px, [s])
            xd = plsc.load_gather(px, [d])
            ys = plsc.load_gather(py, [s])
            yd = plsc.load_gather(py, [d])
            zs = plsc.load_gather(pz, [s])
            zd = plsc.load_gather(pz, [d])
            ts = plsc.load_gather(ty, [s])
            td = plsc.load_gather(ty, [d])
            ddx = xd - xs
            ddy = yd - ys
            ddz = zd - zs
            r2v[sl] = ddx * ddx + ddy * ddy + ddz * ddz
            prv[sl] = td * n_elem + ts
        pltpu.sync_copy(r2v, r2_h.at[row, pl.ds(c0, chunk)])
        pltpu.sync_copy(prv, pr_h.at[row, pl.ds(c0, chunk)])

    mesh = plsc.VectorSubcoreMesh(core_axis_name="c", subcore_axis_name="s")
    fn = pl.kernel(
        body,
        mesh=mesh,
        compiler_params=pltpu.CompilerParams(needs_layout_passes=False),
        out_type=(
            jax.ShapeDtypeStruct((8, cols), jnp.float32),
            jax.ShapeDtypeStruct((8, cols), jnp.int32),
        ),
        scratch_types=[
            pltpu.VMEM((n_atoms,), jnp.float32),
            pltpu.VMEM((n_atoms,), jnp.float32),
            pltpu.VMEM((n_atoms,), jnp.float32),
            pltpu.VMEM((n_atoms,), jnp.int32),
            pltpu.VMEM((chunk,), jnp.int32),
            pltpu.VMEM((chunk,), jnp.int32),
            pltpu.VMEM((chunk,), jnp.float32),
            pltpu.VMEM((chunk,), jnp.int32),
            pltpu.SemaphoreType.DMA,
        ],
    )
    return fn(posx, posy, posz, typ, ei)


_CB = 4096
_BMAX = 10


def _tc_energy(r2p, prp, w1k, w2k, b2k, w3k, b3c, n_edges, ne):
    cols = r2p.shape[1]
    grid = cols // _CB
    nfeat = _BMAX + 2 * ne

    def body(r2_ref, pr_ref, w1_ref, w2_ref, b2_ref, w3_ref, b3_ref, out_ref):
        pid = pl.program_id(0)

        @pl.when(pid == 0)
        def _():
            out_ref[0, 0] = 0.0

        r2 = r2_ref[...]
        pr = pr_ref[...]
        r = jnp.sqrt(r2)
        x = r * (1.0 / _CUTOFF)
        x2 = x * x
        x3 = x2 * x
        x6 = x3 * x3
        x7 = x6 * x
        x8 = x7 * x
        env = 1.0 - 28.0 * x6 + 48.0 * x7 - 21.0 * x8
        env = jnp.where(x < 1.0, env, 0.0)
        scale = jnp.sqrt(2.0 / _CUTOFF) * env / jnp.maximum(r, 1e-12)
        ang = jnp.pi * x
        c1 = jnp.cos(ang)
        two_c1 = c1 + c1
        s = [jnp.sin(ang) * scale]
        sm1 = jnp.zeros_like(r2)
        for _n in range(_BMAX - 1):
            s_next = two_c1 * s[-1] - sm1
            sm1 = s[-1]
            s.append(s_next)
        td = pr // ne
        ts = pr - td * ne
        oh = [(td == t).astype(jnp.float32) for t in range(ne)]
        oh += [(ts == t).astype(jnp.float32) for t in range(ne)]
        feat = jnp.concatenate(s + oh, axis=0)
        pre1 = jnp.dot(w1_ref[...], feat,
                       preferred_element_type=jnp.float32)
        h1 = pre1 * jax.nn.sigmoid(pre1)
        pre2 = jnp.dot(w2_ref[...], h1,
                       preferred_element_type=jnp.float32) + b2_ref[...]
        h2 = pre2 * jax.nn.sigmoid(pre2)
        evec = jnp.dot(w3_ref[...], h2,
                       preferred_element_type=jnp.float32)
        jrow = lax.broadcasted_iota(jnp.int32, (8, _CB), 0) * cols
        lane = lax.broadcasted_iota(jnp.int32, (8, _CB), 1)
        eid = jrow + pid * _CB + lane
        e = jnp.where(eid < n_edges, evec + b3_ref[0, 0], 0.0)
        out_ref[0, 0] += 0.5 * jnp.sum(e)

    out = pl.pallas_call(
        body,
        grid=(grid,),
        in_specs=[
            pl.BlockSpec((8, _CB), lambda i: (0, i)),
            pl.BlockSpec((8, _CB), lambda i: (0, i)),
            pl.BlockSpec((128, 8 * nfeat), lambda i: (0, 0)),
            pl.BlockSpec((128, 128), lambda i: (0, 0)),
            pl.BlockSpec((128, 1), lambda i: (0, 0)),
            pl.BlockSpec((8, 128), lambda i: (0, 0)),
            pl.BlockSpec(memory_space=pltpu.SMEM),
        ],
        out_specs=pl.BlockSpec(memory_space=pltpu.SMEM),
        out_shape=jax.ShapeDtypeStruct((1, 1), jnp.float32),
    )(r2p, prp, w1k, w2k, b2k, w3k, b3c)
    return out


def kernel(positions, type_indices, edge_index, emb_table, W1, b1, W2, b2, W3, b3):
    n_edges = edge_index.shape[1]
    blk_edges = 8 * _CB
    e_pad = ((n_edges + blk_edges - 1) // blk_edges) * blk_edges
    ei2 = jnp.pad(edge_index, ((0, 0), (0, e_pad - n_edges)))
    posx = positions[:, 0]
    posy = positions[:, 1]
    posz = positions[:, 2]

    ne = emb_table.shape[0]
    td_dim = emb_table.shape[1]
    hid = W1.shape[0]
    a_d = emb_table @ W1[:, :td_dim].T + b1
    b_s = emb_table @ W1[:, td_dim:2 * td_dim].T
    w18 = jnp.concatenate([W1[:, 2 * td_dim:], a_d.T, b_s.T], axis=1)
    eye8 = jnp.eye(8, dtype=jnp.float32)
    w1k = jnp.kron(w18, eye8)
    w2k = jnp.kron(W2, eye8)
    b2k = jnp.repeat(b2, 8).reshape(8 * hid, 1)
    w3k = jnp.kron(W3, eye8)

    b3c = b3.reshape(1, 1)
    r2p, prp = _sc_edge_gather(posx, posy, posz, type_indices, ei2, ne)
    out = _tc_energy(r2p, prp, w1k, w2k, b2k, w3k, b3c, n_edges, ne)
    return out[0, 0]


# speedup vs baseline: 1.2320x; 1.1482x over previous
import functools

import jax
import jax.numpy as jnp
from jax import lax
from jax.experimental import pallas as pl
from jax.experimental.pallas import tpu as pltpu
from jax.experimental.pallas import tpu_sc as plsc

_CUTOFF = 0.09
_NW = 32
_LANES = 16


def _sc_edge_gather(posx, posy, posz, typ, ei, e_pad, n_elem):
    n_edges = ei.shape[1]
    chunk = e_pad // _NW
    cols = e_pad // 8
    steps = chunk // _LANES
    n_atoms = posx.shape[0]
    e_al = ((n_edges + 127) // 128) * 128
    base_last = e_al - chunk
    assert base_last % 128 == 0 and chunk % 128 == 0
    valid_in_buf = n_edges - base_last

    def body(posx_h, posy_h, posz_h, typ_h, ei_h, r2_h, pr_h,
             px, py, pz, ty, ev, r2v, prv, sem):
        cid = lax.axis_index("c")
        sid = lax.axis_index("s")
        wid = cid * 16 + sid
        base = jnp.minimum(wid * chunk, base_last)
        row = base // cols
        c0 = base - row * cols
        h1 = pltpu.async_copy(posx_h, px, sem)
        h2 = pltpu.async_copy(posy_h, py, sem)
        h3 = pltpu.async_copy(posz_h, pz, sem)
        h4 = pltpu.async_copy(typ_h, ty, sem)
        h5 = pltpu.async_copy(ei_h.at[:, pl.ds(base, chunk)], ev, sem)
        h1.wait()
        h2.wait()
        h3.wait()
        h4.wait()
        h5.wait()

        if e_al > n_edges:
            @pl.when(wid * chunk >= base_last)
            def _zero_tail():
                lane = lax.iota(jnp.int32, _LANES)
                g0 = (valid_in_buf // _LANES) * _LANES
                for g in range(g0, chunk, _LANES):
                    keep = lane < max(0, min(_LANES, valid_in_buf - g))
                    tsl = pl.ds(g, _LANES)
                    ev[0, tsl] = jnp.where(keep, ev[0, tsl], 0)
                    ev[1, tsl] = jnp.where(keep, ev[1, tsl], 0)

        @plsc.parallel_loop(0, steps, step=1, unroll=8)
        def _step(i):
            sl = pl.ds(i * _LANES, _LANES)
            s = ev[0, sl]
            d = ev[1, sl]
            xs = plsc.load_gather(px, [s])
            xd = plsc.load_gather(px, [d])
            ys = plsc.load_gather(py, [s])
            yd = plsc.load_gather(py, [d])
            zs = plsc.load_gather(pz, [s])
            zd = plsc.load_gather(pz, [d])
            ts = plsc.load_gather(ty, [s])
            td = plsc.load_gather(ty, [d])
            ddx = xd - xs
            ddy = yd - ys
            ddz = zd - zs
            r2v[sl] = ddx * ddx + ddy * ddy + ddz * ddz
            prv[sl] = td * n_elem + ts
        pltpu.sync_copy(r2v, r2_h.at[row, pl.ds(c0, chunk)])
        pltpu.sync_copy(prv, pr_h.at[row, pl.ds(c0, chunk)])

    mesh = plsc.VectorSubcoreMesh(core_axis_name="c", subcore_axis_name="s")
    fn = pl.kernel(
        body,
        mesh=mesh,
        compiler_params=pltpu.CompilerParams(needs_layout_passes=False),
        out_type=(
            jax.ShapeDtypeStruct((8, cols), jnp.float32),
            jax.ShapeDtypeStruct((8, cols), jnp.int32),
        ),
        scratch_types=[
            pltpu.VMEM((n_atoms,), jnp.float32),
            pltpu.VMEM((n_atoms,), jnp.float32),
            pltpu.VMEM((n_atoms,), jnp.float32),
            pltpu.VMEM((n_atoms,), jnp.int32),
            pltpu.VMEM((2, chunk), jnp.int32),
            pltpu.VMEM((chunk,), jnp.float32),
            pltpu.VMEM((chunk,), jnp.int32),
            pltpu.SemaphoreType.DMA,
        ],
    )
    return fn(posx, posy, posz, typ, ei)


_BMAX = 10


def _tc_energy(r2p, prp, w1k, w2k, b2k, w3k, b3c, n_edges, ne):
    cols = r2p.shape[1]
    cb = cols // 8
    grid = cols // cb
    nfeat = _BMAX + 2 * ne

    def body(r2_ref, pr_ref, w1_ref, w2_ref, b2_ref, w3_ref, b3_ref, out_ref):
        pid = pl.program_id(0)

        @pl.when(pid == 0)
        def _():
            out_ref[0, 0] = 0.0

        r2 = r2_ref[...]
        pr = pr_ref[...]
        r = jnp.sqrt(r2)
        x = r * (1.0 / _CUTOFF)
        x2 = x * x
        x3 = x2 * x
        x6 = x3 * x3
        x7 = x6 * x
        x8 = x7 * x
        env = 1.0 - 28.0 * x6 + 48.0 * x7 - 21.0 * x8
        env = jnp.where(x < 1.0, env, 0.0)
        scale = jnp.sqrt(2.0 / _CUTOFF) * env / jnp.maximum(r, 1e-12)
        ang = jnp.pi * x
        c1 = jnp.cos(ang)
        two_c1 = c1 + c1
        s = [jnp.sin(ang) * scale]
        sm1 = jnp.zeros_like(r2)
        for _n in range(_BMAX - 1):
            s_next = two_c1 * s[-1] - sm1
            sm1 = s[-1]
            s.append(s_next)
        td = pr // ne
        ts = pr - td * ne
        oh = [(td == t).astype(jnp.float32) for t in range(ne)]
        oh += [(ts == t).astype(jnp.float32) for t in range(ne)]
        feat = jnp.concatenate(s + oh, axis=0)
        pre1 = jnp.dot(w1_ref[...], feat,
                       preferred_element_type=jnp.float32)
        h1 = pre1 * jax.nn.sigmoid(pre1)
        pre2 = jnp.dot(w2_ref[...], h1,
                       preferred_element_type=jnp.float32) + b2_ref[...]
        h2 = pre2 * jax.nn.sigmoid(pre2)
        evec = jnp.dot(w3_ref[...], h2,
                       preferred_element_type=jnp.float32)
        jrow = lax.broadcasted_iota(jnp.int32, (8, cb), 0) * cols
        lane = lax.broadcasted_iota(jnp.int32, (8, cb), 1)
        eid = jrow + pid * cb + lane
        e = jnp.where(eid < n_edges, evec + b3_ref[0, 0], 0.0)
        out_ref[0, 0] += 0.5 * jnp.sum(e)

    out = pl.pallas_call(
        body,
        grid=(grid,),
        in_specs=[
            pl.BlockSpec((8, cb), lambda i: (0, i)),
            pl.BlockSpec((8, cb), lambda i: (0, i)),
            pl.BlockSpec((128, 8 * nfeat), lambda i: (0, 0)),
            pl.BlockSpec((128, 128), lambda i: (0, 0)),
            pl.BlockSpec((128, 1), lambda i: (0, 0)),
            pl.BlockSpec((8, 128), lambda i: (0, 0)),
            pl.BlockSpec(memory_space=pltpu.SMEM),
        ],
        out_specs=pl.BlockSpec(memory_space=pltpu.SMEM),
        out_shape=jax.ShapeDtypeStruct((1, 1), jnp.float32),
    )(r2p, prp, w1k, w2k, b2k, w3k, b3c)
    return out


def kernel(positions, type_indices, edge_index, emb_table, W1, b1, W2, b2, W3, b3):
    n_edges = edge_index.shape[1]
    e_al = ((n_edges + 127) // 128) * 128
    chunk = ((e_al + _NW * 256 - 1) // (_NW * 256)) * 256
    e_pad = _NW * chunk
    posx = positions[:, 0]
    posy = positions[:, 1]
    posz = positions[:, 2]

    ne = emb_table.shape[0]
    td_dim = emb_table.shape[1]
    hid = W1.shape[0]
    a_d = emb_table @ W1[:, :td_dim].T + b1
    b_s = emb_table @ W1[:, td_dim:2 * td_dim].T
    w18 = jnp.concatenate([W1[:, 2 * td_dim:], a_d.T, b_s.T], axis=1)
    eye8 = jnp.eye(8, dtype=jnp.float32)
    w1k = jnp.kron(w18, eye8)
    w2k = jnp.kron(W2, eye8)
    b2k = jnp.repeat(b2, 8).reshape(8 * hid, 1)
    w3k = jnp.kron(W3, eye8)

    b3c = b3.reshape(1, 1)
    r2p, prp = _sc_edge_gather(posx, posy, posz, type_indices, edge_index,
                               e_pad, ne)
    out = _tc_energy(r2p, prp, w1k, w2k, b2k, w3k, b3c, n_edges, ne)
    return out[0, 0]
